# baseline (speedup 1.0000x reference)
import functools

import jax
import jax.numpy as jnp
from jax import lax
from jax.experimental import pallas as pl
from jax.experimental.pallas import tpu as pltpu
from jax.experimental.pallas import tpu_sc as plsc

_DIM = 2048
_K = 512
_LHID = 1024
_HHID = 8192
_B = 2
_N = 2048
_NTOK = _B * _N

_LIGHT_T = 512
_HEAVY_H = 512


def _gelu(h):
    return 0.5 * h * (1.0 + lax.erf(h * (2.0 ** -0.5)))


def _rms_normed(x, gamma):
    ss = jnp.sum(x * x, axis=-1, keepdims=True)
    norm = jnp.sqrt(ss)
    return x / jnp.clip(norm, 1e-12) * (_DIM ** 0.5) * gamma


def _light_body(x_ref, rt_ref, g_ref, w1_ref, b1_ref, w2_ref, b2_ref,
                out_ref, sc_ref):
    x = x_ref[...]
    sc_ref[...] = jnp.dot(x, rt_ref[...],
                          preferred_element_type=jnp.float32)[:, 0]
    normed = _rms_normed(x, g_ref[...])
    h = jnp.dot(normed, w1_ref[...], preferred_element_type=jnp.float32)
    h = _gelu(h + b1_ref[...])
    out_ref[...] = (jnp.dot(h, w2_ref[...], preferred_element_type=jnp.float32)
                    + b2_ref[...])


def _light_call(xf, rt2, gamma, w1, b1, w2, b2):
    grid = _NTOK // _LIGHT_T
    return pl.pallas_call(
        _light_body,
        grid=(grid,),
        in_specs=[
            pl.BlockSpec((_LIGHT_T, _DIM), lambda i: (i, 0)),
            pl.BlockSpec((_DIM, 1), lambda i: (0, 0)),
            pl.BlockSpec((_DIM,), lambda i: (0,)),
            pl.BlockSpec((_DIM, _LHID), lambda i: (0, 0)),
            pl.BlockSpec((_LHID,), lambda i: (0,)),
            pl.BlockSpec((_LHID, _DIM), lambda i: (0, 0)),
            pl.BlockSpec((_DIM,), lambda i: (0,)),
        ],
        out_specs=[
            pl.BlockSpec((_LIGHT_T, _DIM), lambda i: (i, 0)),
            pl.BlockSpec((_LIGHT_T,), lambda i: (i,)),
        ],
        out_shape=[
            jax.ShapeDtypeStruct((_NTOK, _DIM), jnp.float32),
            jax.ShapeDtypeStruct((_NTOK,), jnp.float32),
        ],
        compiler_params=pltpu.CompilerParams(
            dimension_semantics=("arbitrary",)),
    )(xf, rt2, gamma, w1, b1, w2, b2)


def _heavy_body(xg_ref, lg_ref, g_ref, w1_ref, b1_ref, w2_ref, b2_ref,
                out_ref, normed_ref):
    k = pl.program_id(0)

    @pl.when(k == 0)
    def _():
        normed_ref[...] = _rms_normed(xg_ref[...],
                                      g_ref[...]).astype(jnp.bfloat16)
        out_ref[...] = lg_ref[...] + b2_ref[...]

    h = jnp.dot(normed_ref[...], w1_ref[...].astype(jnp.bfloat16),
                preferred_element_type=jnp.float32)
    h = _gelu(h + b1_ref[...]).astype(jnp.bfloat16)
    out_ref[...] += jnp.dot(h, w2_ref[...].astype(jnp.bfloat16),
                            preferred_element_type=jnp.float32)


def _heavy_call(xg, lightg, gamma, w1, b1, w2, b2):
    grid = _HHID // _HEAVY_H
    nrows = _B * _K
    return pl.pallas_call(
        _heavy_body,
        grid=(grid,),
        in_specs=[
            pl.BlockSpec((nrows, _DIM), lambda k: (0, 0)),
            pl.BlockSpec((nrows, _DIM), lambda k: (0, 0)),
            pl.BlockSpec((_DIM,), lambda k: (0,)),
            pl.BlockSpec((_DIM, _HEAVY_H), lambda k: (0, k)),
            pl.BlockSpec((_HEAVY_H,), lambda k: (k,)),
            pl.BlockSpec((_HEAVY_H, _DIM), lambda k: (k, 0)),
            pl.BlockSpec((_DIM,), lambda k: (0,)),
        ],
        out_specs=pl.BlockSpec((nrows, _DIM), lambda k: (0, 0)),
        out_shape=jax.ShapeDtypeStruct((nrows, _DIM), jnp.float32),
        scratch_shapes=[pltpu.VMEM((nrows, _DIM), jnp.bfloat16)],
        compiler_params=pltpu.CompilerParams(
            dimension_semantics=("arbitrary",)),
    )(xg, lightg, gamma, w1, b1, w2, b2)


_GROWS = _K // 16


def _route_body(scores_hbm, x_hbm, light_hbm, selidx_hbm, gath_hbm, lg_hbm,
                scores_v, keys_v, sel_v, idx_v, rows_v, sel_s, sem):
    c = lax.axis_index("c")
    s = lax.axis_index("s")

    @pl.when(s == 0)
    def _():
        pltpu.sync_copy(scores_hbm.at[pl.ds(c * _N, _N)], scores_v)

        def mk(i, _):
            v = scores_v[pl.ds(i * 16, 16)]
            u = plsc.bitcast(v, jnp.int32)
            neg = lax.shift_right_arithmetic(u, 31)
            key = lax.bitwise_xor(
                u, lax.bitwise_or(neg, jnp.int32(-(2 ** 31))))
            keys_v[pl.ds(i * 16, 16)] = plsc.bitcast(key, jnp.uint32)
            return 0
        lax.fori_loop(0, _N // 16, mk, 0, unroll=4)

        def count_ge(cand):
            def cnt(i, acc):
                k16 = keys_v[pl.ds(i * 16, 16)]
                m = k16 >= cand
                return acc + plsc.all_reduce_population_count(m)
            return lax.fori_loop(0, _N // 16, cnt, jnp.zeros((16,), jnp.int32),
                                 unroll=4)

        def bit_step(b, thr):
            bit = jnp.left_shift(jnp.uint32(1), jnp.uint32(31) - b.astype(jnp.uint32))
            cand = lax.bitwise_or(thr, jnp.broadcast_to(bit, (16,)))
            cnt = count_ge(cand)
            take = cnt >= _K
            return jnp.where(take, cand, thr)

        thr = lax.fori_loop(0, 32, bit_step, jnp.zeros((16,), jnp.uint32))

        def sel_pass(strict, off0):
            def step(t, off):
                i = _N // 16 - 1 - t
                k16 = keys_v[pl.ds(i * 16, 16)]
                k16 = lax.rev(k16, (0,))
                idx = (jnp.full((16,), i * 16 + 15, jnp.int32)
                       - lax.iota(jnp.int32, 16))
                if strict:
                    m = k16 > thr
                else:
                    m = k16 == thr
                rank = plsc.cumsum(jnp.where(m, 1, 0)) - 1
                keep = jnp.logical_and(m, (off + rank) < _K)
                plsc.store_compressed(sel_v.at[pl.ds(off, 16)],
                                      idx + c * _N, mask=keep)
                npick = plsc.all_reduce_population_count(keep)
                return off + jnp.max(npick)
            return lax.fori_loop(0, _N // 16, step, off0)

        off = sel_pass(True, jnp.int32(0))
        sel_pass(False, off)

        pltpu.sync_copy(sel_v.at[pl.ds(0, _K)], sel_s)
        pltpu.sync_copy(sel_v.at[pl.ds(0, _K)],
                        selidx_hbm.at[pl.ds(c * _K, _K)])

    plsc.subcore_barrier()
    pltpu.sync_copy(sel_s.at[pl.ds(s * _GROWS, _GROWS)], idx_v)
    pltpu.async_copy(x_hbm.at[idx_v], rows_v, sem).wait()
    pltpu.sync_copy(rows_v,
                    gath_hbm.at[pl.ds(c * _K + s * _GROWS, _GROWS)])
    pltpu.async_copy(light_hbm.at[idx_v], rows_v, sem).wait()
    pltpu.sync_copy(rows_v,
                    lg_hbm.at[pl.ds(c * _K + s * _GROWS, _GROWS)])


def _route_call(scores, xf, light):
    mesh = plsc.VectorSubcoreMesh(core_axis_name="c", subcore_axis_name="s")
    f = pl.kernel(
        _route_body,
        out_type=[
            jax.ShapeDtypeStruct((_B * _K,), jnp.int32),
            jax.ShapeDtypeStruct((_B * _K, _DIM), jnp.float32),
            jax.ShapeDtypeStruct((_B * _K, _DIM), jnp.float32),
        ],
        mesh=mesh,
        scratch_types=[
            pltpu.VMEM((_N,), jnp.float32),
            pltpu.VMEM((_N,), jnp.uint32),
            pltpu.VMEM((_K + 16,), jnp.int32),
            pltpu.VMEM((_GROWS,), jnp.int32),
            pltpu.VMEM((_GROWS, _DIM), jnp.float32),
            pltpu.VMEM_SHARED((_K,), jnp.int32),
            pltpu.SemaphoreType.DMA,
        ],
        compiler_params=pltpu.CompilerParams(needs_layout_passes=False),
    )
    return f(scores, xf, light)


_CROWS = _B * _K // 32
_CHALF = _CROWS // 2


def _scatter_body(heavy_hbm, selidx_hbm, out_ref, idx_v, rows_v, sem):
    c = lax.axis_index("c")
    s = lax.axis_index("s")
    base = (s * 2 + c) * _CROWS
    pltpu.sync_copy(selidx_hbm.at[pl.ds(base, _CROWS)], idx_v)
    pltpu.sync_copy(heavy_hbm.at[pl.ds(base, _CROWS)], rows_v)
    pltpu.async_copy(rows_v, out_ref.at[idx_v], sem).wait()


def _scatter_call(heavy, selidx, out_ref):
    mesh = plsc.VectorSubcoreMesh(core_axis_name="c", subcore_axis_name="s")
    f = pl.kernel(
        _scatter_body,
        out_type=[],
        mesh=mesh,
        scratch_types=[
            pltpu.VMEM((_CROWS,), jnp.int32),
            pltpu.VMEM((_CROWS, _DIM), jnp.float32),
            pltpu.SemaphoreType.DMA,
        ],
        compiler_params=pltpu.CompilerParams(needs_layout_passes=False),
    )
    f(heavy, selidx, out_ref)


def kernel(x, routing_token, gamma_l, w1_l, b1_l, w2_l, b2_l,
           gamma_h, w1_h, b1_h, w2_h, b2_h):
    xf = x.reshape(_NTOK, _DIM)
    rt2 = routing_token.reshape(_DIM, 1)
    light, scores = _light_call(xf, rt2, gamma_l, w1_l, b1_l, w2_l, b2_l)
    selidx, gathered, lightg = _route_call(scores, xf, light)
    heavy = _heavy_call(gathered, lightg, gamma_h, w1_h, b1_h, w2_h, b2_h)
    ref = jax.new_ref(light)
    _scatter_call(heavy, selidx, ref)
    return jax.freeze(ref).reshape(_B, _N, _DIM)


# speedup vs baseline: 1.0119x; 1.0119x over previous
import functools

import jax
import jax.numpy as jnp
from jax import lax
from jax.experimental import pallas as pl
from jax.experimental.pallas import tpu as pltpu
from jax.experimental.pallas import tpu_sc as plsc

_DIM = 2048
_K = 512
_LHID = 1024
_HHID = 8192
_B = 2
_N = 2048
_NTOK = _B * _N

_LIGHT_T = 512
_HEAVY_H = 512


def _gelu(h):
    return 0.5 * h * (1.0 + lax.erf(h * (2.0 ** -0.5)))


def _rms_normed(x, gamma):
    ss = jnp.sum(x * x, axis=-1, keepdims=True)
    norm = jnp.sqrt(ss)
    return x / jnp.clip(norm, 1e-12) * (_DIM ** 0.5) * gamma


def _light_body(x_ref, rt_ref, g_ref, w1_ref, b1_ref, w2_ref, b2_ref,
                out_ref, sc_ref):
    x = x_ref[...]
    sc_ref[...] = jnp.dot(x, rt_ref[...],
                          preferred_element_type=jnp.float32)[:, 0]
    normed = _rms_normed(x, g_ref[...])
    h = jnp.dot(normed, w1_ref[...], preferred_element_type=jnp.float32)
    h = _gelu(h + b1_ref[...])
    out_ref[...] = (jnp.dot(h, w2_ref[...], preferred_element_type=jnp.float32)
                    + b2_ref[...])


def _light_call(xf, rt2, gamma, w1, b1, w2, b2):
    grid = _NTOK // _LIGHT_T
    return pl.pallas_call(
        _light_body,
        grid=(grid,),
        in_specs=[
            pl.BlockSpec((_LIGHT_T, _DIM), lambda i: (i, 0)),
            pl.BlockSpec((_DIM, 1), lambda i: (0, 0)),
            pl.BlockSpec((_DIM,), lambda i: (0,)),
            pl.BlockSpec((_DIM, _LHID), lambda i: (0, 0)),
            pl.BlockSpec((_LHID,), lambda i: (0,)),
            pl.BlockSpec((_LHID, _DIM), lambda i: (0, 0)),
            pl.BlockSpec((_DIM,), lambda i: (0,)),
        ],
        out_specs=[
            pl.BlockSpec((_LIGHT_T, _DIM), lambda i: (i, 0)),
            pl.BlockSpec((_LIGHT_T,), lambda i: (i,)),
        ],
        out_shape=[
            jax.ShapeDtypeStruct((_NTOK, _DIM), jnp.float32),
            jax.ShapeDtypeStruct((_NTOK,), jnp.float32),
        ],
        compiler_params=pltpu.CompilerParams(
            dimension_semantics=("arbitrary",)),
    )(xf, rt2, gamma, w1, b1, w2, b2)


def _heavy_body(xg_ref, lg_ref, g_ref, w1_ref, b1_ref, w2_ref, b2_ref,
                out_ref, normed_ref):
    k = pl.program_id(0)

    @pl.when(k == 0)
    def _():
        normed_ref[...] = _rms_normed(xg_ref[...], g_ref[...])
        out_ref[...] = lg_ref[...] + b2_ref[...]

    h = jnp.dot(normed_ref[...], w1_ref[...],
                preferred_element_type=jnp.float32)
    h = _gelu(h + b1_ref[...])
    out_ref[...] += jnp.dot(h, w2_ref[...],
                            preferred_element_type=jnp.float32)


def _heavy_call(xg, lightg, gamma, w1, b1, w2, b2):
    grid = _HHID // _HEAVY_H
    nrows = _B * _K
    return pl.pallas_call(
        _heavy_body,
        grid=(grid,),
        in_specs=[
            pl.BlockSpec((nrows, _DIM), lambda k: (0, 0)),
            pl.BlockSpec((nrows, _DIM), lambda k: (0, 0)),
            pl.BlockSpec((_DIM,), lambda k: (0,)),
            pl.BlockSpec((_DIM, _HEAVY_H), lambda k: (0, k)),
            pl.BlockSpec((_HEAVY_H,), lambda k: (k,)),
            pl.BlockSpec((_HEAVY_H, _DIM), lambda k: (k, 0)),
            pl.BlockSpec((_DIM,), lambda k: (0,)),
        ],
        out_specs=pl.BlockSpec((nrows, _DIM), lambda k: (0, 0)),
        out_shape=jax.ShapeDtypeStruct((nrows, _DIM), jnp.float32),
        scratch_shapes=[pltpu.VMEM((nrows, _DIM), jnp.float32)],
        compiler_params=pltpu.CompilerParams(
            dimension_semantics=("arbitrary",)),
    )(xg, lightg, gamma, w1, b1, w2, b2)


_GROWS = _K // 16


def _route_body(scores_hbm, x_hbm, light_hbm, selidx_hbm, gath_hbm, lg_hbm,
                scores_v, keys_v, sel_v, idx_v, rows_v, sel_s, sem):
    c = lax.axis_index("c")
    s = lax.axis_index("s")

    @pl.when(s == 0)
    def _():
        pltpu.sync_copy(scores_hbm.at[pl.ds(c * _N, _N)], scores_v)

        def mk(i, _):
            v = scores_v[pl.ds(i * 16, 16)]
            u = plsc.bitcast(v, jnp.int32)
            neg = lax.shift_right_arithmetic(u, 31)
            key = lax.bitwise_xor(
                u, lax.bitwise_or(neg, jnp.int32(-(2 ** 31))))
            keys_v[pl.ds(i * 16, 16)] = plsc.bitcast(key, jnp.uint32)
            return 0
        lax.fori_loop(0, _N // 16, mk, 0, unroll=4)

        def count_ge(cand):
            def cnt(i, acc):
                k16 = keys_v[pl.ds(i * 16, 16)]
                m = k16 >= cand
                return acc + plsc.all_reduce_population_count(m)
            return lax.fori_loop(0, _N // 16, cnt, jnp.zeros((16,), jnp.int32),
                                 unroll=4)

        def bit_step(b, thr):
            bit = jnp.left_shift(jnp.uint32(1), jnp.uint32(31) - b.astype(jnp.uint32))
            cand = lax.bitwise_or(thr, jnp.broadcast_to(bit, (16,)))
            cnt = count_ge(cand)
            take = cnt >= _K
            return jnp.where(take, cand, thr)

        thr = lax.fori_loop(0, 32, bit_step, jnp.zeros((16,), jnp.uint32))

        def sel_pass(strict, off0):
            def step(t, off):
                i = _N // 16 - 1 - t
                k16 = keys_v[pl.ds(i * 16, 16)]
                k16 = lax.rev(k16, (0,))
                idx = (jnp.full((16,), i * 16 + 15, jnp.int32)
                       - lax.iota(jnp.int32, 16))
                if strict:
                    m = k16 > thr
                else:
                    m = k16 == thr
                rank = plsc.cumsum(jnp.where(m, 1, 0)) - 1
                keep = jnp.logical_and(m, (off + rank) < _K)
                plsc.store_compressed(sel_v.at[pl.ds(off, 16)],
                                      idx + c * _N, mask=keep)
                npick = plsc.all_reduce_population_count(keep)
                return off + jnp.max(npick)
            return lax.fori_loop(0, _N // 16, step, off0)

        off = sel_pass(True, jnp.int32(0))
        sel_pass(False, off)

        pltpu.sync_copy(sel_v.at[pl.ds(0, _K)], sel_s)
        pltpu.sync_copy(sel_v.at[pl.ds(0, _K)],
                        selidx_hbm.at[pl.ds(c * _K, _K)])

    plsc.subcore_barrier()
    pltpu.sync_copy(sel_s.at[pl.ds(s * _GROWS, _GROWS)], idx_v)
    pltpu.async_copy(x_hbm.at[idx_v], rows_v, sem).wait()
    pltpu.sync_copy(rows_v,
                    gath_hbm.at[pl.ds(c * _K + s * _GROWS, _GROWS)])
    pltpu.async_copy(light_hbm.at[idx_v], rows_v, sem).wait()
    pltpu.sync_copy(rows_v,
                    lg_hbm.at[pl.ds(c * _K + s * _GROWS, _GROWS)])


def _route_call(scores, xf, light):
    mesh = plsc.VectorSubcoreMesh(core_axis_name="c", subcore_axis_name="s")
    f = pl.kernel(
        _route_body,
        out_type=[
            jax.ShapeDtypeStruct((_B * _K,), jnp.int32),
            jax.ShapeDtypeStruct((_B * _K, _DIM), jnp.float32),
            jax.ShapeDtypeStruct((_B * _K, _DIM), jnp.float32),
        ],
        mesh=mesh,
        scratch_types=[
            pltpu.VMEM((_N,), jnp.float32),
            pltpu.VMEM((_N,), jnp.uint32),
            pltpu.VMEM((_K + 16,), jnp.int32),
            pltpu.VMEM((_GROWS,), jnp.int32),
            pltpu.VMEM((_GROWS, _DIM), jnp.float32),
            pltpu.VMEM_SHARED((_K,), jnp.int32),
            pltpu.SemaphoreType.DMA,
        ],
        compiler_params=pltpu.CompilerParams(needs_layout_passes=False),
    )
    return f(scores, xf, light)


_CROWS = _B * _K // 32
_CHALF = _CROWS // 2


def _scatter_body(heavy_hbm, selidx_hbm, out_ref, idx_v, rows_v, sem):
    c = lax.axis_index("c")
    s = lax.axis_index("s")
    base = (s * 2 + c) * _CROWS
    pltpu.sync_copy(selidx_hbm.at[pl.ds(base, _CROWS)], idx_v)
    pltpu.sync_copy(heavy_hbm.at[pl.ds(base, _CROWS)], rows_v)
    pltpu.async_copy(rows_v, out_ref.at[idx_v], sem).wait()


def _scatter_call(heavy, selidx, out_ref):
    mesh = plsc.VectorSubcoreMesh(core_axis_name="c", subcore_axis_name="s")
    f = pl.kernel(
        _scatter_body,
        out_type=[],
        mesh=mesh,
        scratch_types=[
            pltpu.VMEM((_CROWS,), jnp.int32),
            pltpu.VMEM((_CROWS, _DIM), jnp.float32),
            pltpu.SemaphoreType.DMA,
        ],
        compiler_params=pltpu.CompilerParams(needs_layout_passes=False),
    )
    f(heavy, selidx, out_ref)


def kernel(x, routing_token, gamma_l, w1_l, b1_l, w2_l, b2_l,
           gamma_h, w1_h, b1_h, w2_h, b2_h):
    xf = x.reshape(_NTOK, _DIM)
    rt2 = routing_token.reshape(_DIM, 1)
    light, scores = _light_call(xf, rt2, gamma_l, w1_l, b1_l, w2_l, b2_l)
    selidx, gathered, lightg = _route_call(scores, xf, light)
    heavy = _heavy_call(gathered, lightg, gamma_h, w1_h, b1_h, w2_h, b2_h)
    ref = jax.new_ref(light)
    _scatter_call(heavy, selidx, ref)
    return jax.freeze(ref).reshape(_B, _N, _DIM)


# speedup vs baseline: 1.0123x; 1.0004x over previous
import functools

import jax
import jax.numpy as jnp
from jax import lax
from jax.experimental import pallas as pl
from jax.experimental.pallas import tpu as pltpu
from jax.experimental.pallas import tpu_sc as plsc

_DIM = 2048
_K = 512
_LHID = 1024
_HHID = 8192
_B = 2
_N = 2048
_NTOK = _B * _N

_LIGHT_T = 512
_HEAVY_H = 1024


def _gelu(h):
    return 0.5 * h * (1.0 + lax.erf(h * (2.0 ** -0.5)))


def _rms_normed(x, gamma):
    ss = jnp.sum(x * x, axis=-1, keepdims=True)
    norm = jnp.sqrt(ss)
    return x / jnp.clip(norm, 1e-12) * (_DIM ** 0.5) * gamma


def _light_body(x_ref, rt_ref, g_ref, w1_ref, b1_ref, w2_ref, b2_ref,
                out_ref, sc_ref):
    x = x_ref[...]
    sc_ref[...] = jnp.dot(x, rt_ref[...],
                          preferred_element_type=jnp.float32)[:, 0]
    normed = _rms_normed(x, g_ref[...])
    h = jnp.dot(normed, w1_ref[...], preferred_element_type=jnp.float32)
    h = _gelu(h + b1_ref[...])
    out_ref[...] = (jnp.dot(h, w2_ref[...], preferred_element_type=jnp.float32)
                    + b2_ref[...])


def _light_call(xf, rt2, gamma, w1, b1, w2, b2):
    grid = _NTOK // _LIGHT_T
    return pl.pallas_call(
        _light_body,
        grid=(grid,),
        in_specs=[
            pl.BlockSpec((_LIGHT_T, _DIM), lambda i: (i, 0)),
            pl.BlockSpec((_DIM, 1), lambda i: (0, 0)),
            pl.BlockSpec((_DIM,), lambda i: (0,)),
            pl.BlockSpec((_DIM, _LHID), lambda i: (0, 0)),
            pl.BlockSpec((_LHID,), lambda i: (0,)),
            pl.BlockSpec((_LHID, _DIM), lambda i: (0, 0)),
            pl.BlockSpec((_DIM,), lambda i: (0,)),
        ],
        out_specs=[
            pl.BlockSpec((_LIGHT_T, _DIM), lambda i: (i, 0)),
            pl.BlockSpec((_LIGHT_T,), lambda i: (i,)),
        ],
        out_shape=[
            jax.ShapeDtypeStruct((_NTOK, _DIM), jnp.float32),
            jax.ShapeDtypeStruct((_NTOK,), jnp.float32),
        ],
        compiler_params=pltpu.CompilerParams(
            dimension_semantics=("arbitrary",)),
    )(xf, rt2, gamma, w1, b1, w2, b2)


def _heavy_body(xg_ref, lg_ref, g_ref, w1_ref, b1_ref, w2_ref, b2_ref,
                out_ref, buf_ref, sem1, sem2):
    k = pl.program_id(0)

    @pl.when(k == 0)
    def _():
        cp_x = pltpu.make_async_copy(xg_ref, buf_ref, sem1)
        cp_l = pltpu.make_async_copy(lg_ref, out_ref, sem2)
        cp_x.start()
        cp_l.start()
        cp_x.wait()
        buf_ref[...] = _rms_normed(buf_ref[...], g_ref[...])
        cp_l.wait()
        out_ref[...] += b2_ref[...]

    h = jnp.dot(buf_ref[...], w1_ref[...],
                preferred_element_type=jnp.float32)
    h = _gelu(h + b1_ref[...])
    out_ref[...] += jnp.dot(h, w2_ref[...],
                            preferred_element_type=jnp.float32)


def _heavy_call(xg, lightg, gamma, w1, b1, w2, b2):
    grid = _HHID // _HEAVY_H
    nrows = _B * _K
    return pl.pallas_call(
        _heavy_body,
        grid=(grid,),
        in_specs=[
            pl.BlockSpec(memory_space=pl.ANY),
            pl.BlockSpec(memory_space=pl.ANY),
            pl.BlockSpec((_DIM,), lambda k: (0,)),
            pl.BlockSpec((_DIM, _HEAVY_H), lambda k: (0, k)),
            pl.BlockSpec((_HEAVY_H,), lambda k: (k,)),
            pl.BlockSpec((_HEAVY_H, _DIM), lambda k: (k, 0)),
            pl.BlockSpec((_DIM,), lambda k: (0,)),
        ],
        out_specs=pl.BlockSpec((nrows, _DIM), lambda k: (0, 0)),
        out_shape=jax.ShapeDtypeStruct((nrows, _DIM), jnp.float32),
        scratch_shapes=[
            pltpu.VMEM((nrows, _DIM), jnp.float32),
            pltpu.SemaphoreType.DMA,
            pltpu.SemaphoreType.DMA,
        ],
        compiler_params=pltpu.CompilerParams(
            dimension_semantics=("arbitrary",)),
    )(xg, lightg, gamma, w1, b1, w2, b2)


_GROWS = _K // 16


def _route_body(scores_hbm, x_hbm, light_hbm, selidx_hbm, gath_hbm, lg_hbm,
                scores_v, keys_v, sel_v, idx_v, rows_v, sel_s, sem):
    c = lax.axis_index("c")
    s = lax.axis_index("s")

    @pl.when(s == 0)
    def _():
        pltpu.sync_copy(scores_hbm.at[pl.ds(c * _N, _N)], scores_v)

        def mk(i, _):
            v = scores_v[pl.ds(i * 16, 16)]
            u = plsc.bitcast(v, jnp.int32)
            neg = lax.shift_right_arithmetic(u, 31)
            key = lax.bitwise_xor(
                u, lax.bitwise_or(neg, jnp.int32(-(2 ** 31))))
            keys_v[pl.ds(i * 16, 16)] = plsc.bitcast(key, jnp.uint32)
            return 0
        lax.fori_loop(0, _N // 16, mk, 0, unroll=4)

        def count_ge(cand):
            def cnt(i, acc):
                k16 = keys_v[pl.ds(i * 16, 16)]
                m = k16 >= cand
                return acc + plsc.all_reduce_population_count(m)
            return lax.fori_loop(0, _N // 16, cnt, jnp.zeros((16,), jnp.int32),
                                 unroll=4)

        def bit_step(b, thr):
            bit = jnp.left_shift(jnp.uint32(1), jnp.uint32(31) - b.astype(jnp.uint32))
            cand = lax.bitwise_or(thr, jnp.broadcast_to(bit, (16,)))
            cnt = count_ge(cand)
            take = cnt >= _K
            return jnp.where(take, cand, thr)

        thr = lax.fori_loop(0, 32, bit_step, jnp.zeros((16,), jnp.uint32))

        def sel_pass(strict, off0):
            def step(t, off):
                i = _N // 16 - 1 - t
                k16 = keys_v[pl.ds(i * 16, 16)]
                k16 = lax.rev(k16, (0,))
                idx = (jnp.full((16,), i * 16 + 15, jnp.int32)
                       - lax.iota(jnp.int32, 16))
                if strict:
                    m = k16 > thr
                else:
                    m = k16 == thr
                rank = plsc.cumsum(jnp.where(m, 1, 0)) - 1
                keep = jnp.logical_and(m, (off + rank) < _K)
                plsc.store_compressed(sel_v.at[pl.ds(off, 16)],
                                      idx + c * _N, mask=keep)
                npick = plsc.all_reduce_population_count(keep)
                return off + jnp.max(npick)
            return lax.fori_loop(0, _N // 16, step, off0)

        off = sel_pass(True, jnp.int32(0))
        sel_pass(False, off)

        pltpu.sync_copy(sel_v.at[pl.ds(0, _K)], sel_s)
        pltpu.sync_copy(sel_v.at[pl.ds(0, _K)],
                        selidx_hbm.at[pl.ds(c * _K, _K)])

    plsc.subcore_barrier()
    pltpu.sync_copy(sel_s.at[pl.ds(s * _GROWS, _GROWS)], idx_v)
    pltpu.async_copy(x_hbm.at[idx_v], rows_v, sem).wait()
    pltpu.sync_copy(rows_v,
                    gath_hbm.at[pl.ds(c * _K + s * _GROWS, _GROWS)])
    pltpu.async_copy(light_hbm.at[idx_v], rows_v, sem).wait()
    pltpu.sync_copy(rows_v,
                    lg_hbm.at[pl.ds(c * _K + s * _GROWS, _GROWS)])


def _route_call(scores, xf, light):
    mesh = plsc.VectorSubcoreMesh(core_axis_name="c", subcore_axis_name="s")
    f = pl.kernel(
        _route_body,
        out_type=[
            jax.ShapeDtypeStruct((_B * _K,), jnp.int32),
            jax.ShapeDtypeStruct((_B * _K, _DIM), jnp.float32),
            jax.ShapeDtypeStruct((_B * _K, _DIM), jnp.float32),
        ],
        mesh=mesh,
        scratch_types=[
            pltpu.VMEM((_N,), jnp.float32),
            pltpu.VMEM((_N,), jnp.uint32),
            pltpu.VMEM((_K + 16,), jnp.int32),
            pltpu.VMEM((_GROWS,), jnp.int32),
            pltpu.VMEM((_GROWS, _DIM), jnp.float32),
            pltpu.VMEM_SHARED((_K,), jnp.int32),
            pltpu.SemaphoreType.DMA,
        ],
        compiler_params=pltpu.CompilerParams(needs_layout_passes=False),
    )
    return f(scores, xf, light)


_CROWS = _B * _K // 32
_CHALF = _CROWS // 2


def _scatter_body(heavy_hbm, selidx_hbm, out_ref, idx_v, rows_v, sem):
    c = lax.axis_index("c")
    s = lax.axis_index("s")
    base = (s * 2 + c) * _CROWS
    pltpu.sync_copy(selidx_hbm.at[pl.ds(base, _CROWS)], idx_v)
    pltpu.sync_copy(heavy_hbm.at[pl.ds(base, _CROWS)], rows_v)
    pltpu.async_copy(rows_v, out_ref.at[idx_v], sem).wait()


def _scatter_call(heavy, selidx, out_ref):
    mesh = plsc.VectorSubcoreMesh(core_axis_name="c", subcore_axis_name="s")
    f = pl.kernel(
        _scatter_body,
        out_type=[],
        mesh=mesh,
        scratch_types=[
            pltpu.VMEM((_CROWS,), jnp.int32),
            pltpu.VMEM((_CROWS, _DIM), jnp.float32),
            pltpu.SemaphoreType.DMA,
        ],
        compiler_params=pltpu.CompilerParams(needs_layout_passes=False),
    )
    f(heavy, selidx, out_ref)


def kernel(x, routing_token, gamma_l, w1_l, b1_l, w2_l, b2_l,
           gamma_h, w1_h, b1_h, w2_h, b2_h):
    xf = x.reshape(_NTOK, _DIM)
    rt2 = routing_token.reshape(_DIM, 1)
    light, scores = _light_call(xf, rt2, gamma_l, w1_l, b1_l, w2_l, b2_l)
    selidx, gathered, lightg = _route_call(scores, xf, light)
    heavy = _heavy_call(gathered, lightg, gamma_h, w1_h, b1_h, w2_h, b2_h)
    ref = jax.new_ref(light)
    _scatter_call(heavy, selidx, ref)
    return jax.freeze(ref).reshape(_B, _N, _DIM)


# speedup vs baseline: 1.1219x; 1.1082x over previous
import functools

import jax
import jax.numpy as jnp
from jax import lax
from jax.experimental import pallas as pl
from jax.experimental.pallas import tpu as pltpu
from jax.experimental.pallas import tpu_sc as plsc

_DIM = 2048
_K = 512
_LHID = 1024
_HHID = 8192
_B = 2
_N = 2048
_NTOK = _B * _N

_LIGHT_T = 512
_HEAVY_H = 1024


def _gelu(h):
    return 0.5 * h * (1.0 + lax.erf(h * (2.0 ** -0.5)))


def _rms_normed(x, gamma):
    ss = jnp.sum(x * x, axis=-1, keepdims=True)
    norm = jnp.sqrt(ss)
    return x / jnp.clip(norm, 1e-12) * (_DIM ** 0.5) * gamma


def _light_body(x_ref, g_ref, w1_ref, b1_ref, w2_ref, b2_ref, out_ref):
    x = x_ref[...]
    normed = _rms_normed(x, g_ref[...])
    h = jnp.dot(normed, w1_ref[...], preferred_element_type=jnp.float32)
    h = _gelu(h + b1_ref[...])
    out_ref[...] = (jnp.dot(h, w2_ref[...], preferred_element_type=jnp.float32)
                    + b2_ref[...])


def _light_call(xf, gamma, w1, b1, w2, b2):
    grid = _NTOK // _LIGHT_T
    return pl.pallas_call(
        _light_body,
        grid=(grid,),
        in_specs=[
            pl.BlockSpec((_LIGHT_T, _DIM), lambda i: (i, 0)),
            pl.BlockSpec((_DIM,), lambda i: (0,)),
            pl.BlockSpec((_DIM, _LHID), lambda i: (0, 0)),
            pl.BlockSpec((_LHID,), lambda i: (0,)),
            pl.BlockSpec((_LHID, _DIM), lambda i: (0, 0)),
            pl.BlockSpec((_DIM,), lambda i: (0,)),
        ],
        out_specs=pl.BlockSpec((_LIGHT_T, _DIM), lambda i: (i, 0)),
        out_shape=jax.ShapeDtypeStruct((_NTOK, _DIM), jnp.float32),
        compiler_params=pltpu.CompilerParams(
            dimension_semantics=("arbitrary",)),
    )(xf, gamma, w1, b1, w2, b2)


def _heavy_body(xg_ref, lg_ref, g_ref, w1_ref, b1_ref, w2_ref, b2_ref,
                out_ref, buf_ref, sem1, sem2):
    k = pl.program_id(0)

    @pl.when(k == 0)
    def _():
        cp_x = pltpu.make_async_copy(xg_ref, buf_ref, sem1)
        cp_l = pltpu.make_async_copy(lg_ref, out_ref, sem2)
        cp_x.start()
        cp_l.start()
        cp_x.wait()
        buf_ref[...] = _rms_normed(buf_ref[...], g_ref[...])
        cp_l.wait()
        out_ref[...] += b2_ref[...]

    h = jnp.dot(buf_ref[...], w1_ref[...],
                preferred_element_type=jnp.float32)
    h = _gelu(h + b1_ref[...])
    out_ref[...] += jnp.dot(h, w2_ref[...],
                            preferred_element_type=jnp.float32)


def _heavy_call(xg, lightg, gamma, w1, b1, w2, b2):
    grid = _HHID // _HEAVY_H
    nrows = _B * _K
    return pl.pallas_call(
        _heavy_body,
        grid=(grid,),
        in_specs=[
            pl.BlockSpec(memory_space=pl.ANY),
            pl.BlockSpec(memory_space=pl.ANY),
            pl.BlockSpec((_DIM,), lambda k: (0,)),
            pl.BlockSpec((_DIM, _HEAVY_H), lambda k: (0, k)),
            pl.BlockSpec((_HEAVY_H,), lambda k: (k,)),
            pl.BlockSpec((_HEAVY_H, _DIM), lambda k: (k, 0)),
            pl.BlockSpec((_DIM,), lambda k: (0,)),
        ],
        out_specs=pl.BlockSpec((nrows, _DIM), lambda k: (0, 0)),
        out_shape=jax.ShapeDtypeStruct((nrows, _DIM), jnp.float32),
        scratch_shapes=[
            pltpu.VMEM((nrows, _DIM), jnp.float32),
            pltpu.SemaphoreType.DMA,
            pltpu.SemaphoreType.DMA,
        ],
        compiler_params=pltpu.CompilerParams(
            dimension_semantics=("arbitrary",)),
    )(xg, lightg, gamma, w1, b1, w2, b2)


_GROWS = _K // 16


_SC_TOK = 128
_SC_G = 32


def _route_body(x_hbm, rt_hbm, selidx_hbm, gath_hbm,
                rt_v, xb, scv, scores_v, keys_v, sel_v, idx_v,
                scores_s, sel_s, sem):
    c = lax.axis_index("c")
    s = lax.axis_index("s")

    tb = c * _N + s * _SC_TOK
    pltpu.sync_copy(rt_hbm, rt_v)
    for g in range(_SC_TOK // _SC_G):
        pltpu.sync_copy(x_hbm.at[pl.ds(tb + g * _SC_G, _SC_G)], xb)

        def jstep(j, accs):
            rtj = rt_v[pl.ds(j * 16, 16)]
            return tuple(accs[t] + xb[t, pl.ds(j * 16, 16)] * rtj
                         for t in range(_SC_G))
        accs = lax.fori_loop(0, _DIM // 16, jstep,
                             tuple(jnp.zeros((16,), jnp.float32)
                                   for _ in range(_SC_G)))
        lane = lax.iota(jnp.int32, 16)
        for sub in range(_SC_G // 16):
            vec = jnp.zeros((16,), jnp.float32)
            for t in range(16):
                vec = jnp.where(lane == t, jnp.sum(accs[sub * 16 + t]), vec)
            scv[pl.ds(sub * 16, 16)] = vec
        pltpu.sync_copy(scv,
                        scores_s.at[pl.ds(s * _SC_TOK + g * _SC_G, _SC_G)])
    plsc.subcore_barrier()

    @pl.when(s == 0)
    def _():
        pltpu.sync_copy(scores_s, scores_v)

        def mk(i, _):
            v = scores_v[pl.ds(i * 16, 16)]
            u = plsc.bitcast(v, jnp.int32)
            neg = lax.shift_right_arithmetic(u, 31)
            key = lax.bitwise_xor(
                u, lax.bitwise_or(neg, jnp.int32(-(2 ** 31))))
            keys_v[pl.ds(i * 16, 16)] = plsc.bitcast(key, jnp.uint32)
            return 0
        lax.fori_loop(0, _N // 16, mk, 0, unroll=4)

        def count_ge(cand):
            def cnt(i, acc):
                k16 = keys_v[pl.ds(i * 16, 16)]
                m = k16 >= cand
                return acc + plsc.all_reduce_population_count(m)
            return lax.fori_loop(0, _N // 16, cnt, jnp.zeros((16,), jnp.int32),
                                 unroll=4)

        def bit_step(b, thr):
            bit = jnp.left_shift(jnp.uint32(1), jnp.uint32(31) - b.astype(jnp.uint32))
            cand = lax.bitwise_or(thr, jnp.broadcast_to(bit, (16,)))
            cnt = count_ge(cand)
            take = cnt >= _K
            return jnp.where(take, cand, thr)

        thr = lax.fori_loop(0, 32, bit_step, jnp.zeros((16,), jnp.uint32))

        def sel_pass(strict, off0):
            def step(t, off):
                i = _N // 16 - 1 - t
                k16 = keys_v[pl.ds(i * 16, 16)]
                k16 = lax.rev(k16, (0,))
                idx = (jnp.full((16,), i * 16 + 15, jnp.int32)
                       - lax.iota(jnp.int32, 16))
                if strict:
                    m = k16 > thr
                else:
                    m = k16 == thr
                rank = plsc.cumsum(jnp.where(m, 1, 0)) - 1
                keep = jnp.logical_and(m, (off + rank) < _K)
                plsc.store_compressed(sel_v.at[pl.ds(off, 16)],
                                      idx + c * _N, mask=keep)
                npick = plsc.all_reduce_population_count(keep)
                return off + jnp.max(npick)
            return lax.fori_loop(0, _N // 16, step, off0)

        off = sel_pass(True, jnp.int32(0))
        sel_pass(False, off)

        pltpu.sync_copy(sel_v.at[pl.ds(0, _K)], sel_s)
        pltpu.sync_copy(sel_v.at[pl.ds(0, _K)],
                        selidx_hbm.at[pl.ds(c * _K, _K)])

    plsc.subcore_barrier()

    pltpu.sync_copy(sel_s.at[pl.ds(s * _GROWS, _GROWS)], idx_v)
    pltpu.async_copy(x_hbm.at[idx_v], xb, sem).wait()
    pltpu.sync_copy(xb,
                    gath_hbm.at[pl.ds(c * _K + s * _GROWS, _GROWS)])


def _route_call(xf, rt):
    mesh = plsc.VectorSubcoreMesh(core_axis_name="c", subcore_axis_name="s")
    f = pl.kernel(
        _route_body,
        out_type=[
            jax.ShapeDtypeStruct((_B * _K,), jnp.int32),
            jax.ShapeDtypeStruct((_B * _K, _DIM), jnp.float32),
        ],
        mesh=mesh,
        scratch_types=[
            pltpu.VMEM((_DIM,), jnp.float32),
            pltpu.VMEM((_SC_G, _DIM), jnp.float32),
            pltpu.VMEM((_SC_G,), jnp.float32),
            pltpu.VMEM((_N,), jnp.float32),
            pltpu.VMEM((_N,), jnp.uint32),
            pltpu.VMEM((_K + 16,), jnp.int32),
            pltpu.VMEM((_GROWS,), jnp.int32),
            pltpu.VMEM_SHARED((_N,), jnp.float32),
            pltpu.VMEM_SHARED((_K,), jnp.int32),
            pltpu.SemaphoreType.DMA,
        ],
        compiler_params=pltpu.CompilerParams(needs_layout_passes=False),
    )
    return f(xf, rt)


def _lgather_body(light_hbm, selidx_hbm, lg_hbm, idx_v, rows_v, sem):
    c = lax.axis_index("c")
    s = lax.axis_index("s")
    base = (s * 2 + c) * _GROWS
    pltpu.sync_copy(selidx_hbm.at[pl.ds(base, _GROWS)], idx_v)
    pltpu.async_copy(light_hbm.at[idx_v], rows_v, sem).wait()
    pltpu.sync_copy(rows_v, lg_hbm.at[pl.ds(base, _GROWS)])


def _lgather_call(light, selidx):
    mesh = plsc.VectorSubcoreMesh(core_axis_name="c", subcore_axis_name="s")
    f = pl.kernel(
        _lgather_body,
        out_type=jax.ShapeDtypeStruct((_B * _K, _DIM), jnp.float32),
        mesh=mesh,
        scratch_types=[
            pltpu.VMEM((_GROWS,), jnp.int32),
            pltpu.VMEM((_GROWS, _DIM), jnp.float32),
            pltpu.SemaphoreType.DMA,
        ],
        compiler_params=pltpu.CompilerParams(needs_layout_passes=False),
    )
    return f(light, selidx)


_CROWS = _B * _K // 32
_CHALF = _CROWS // 2


def _scatter_body(heavy_hbm, selidx_hbm, out_ref, idx_v, rows_v, sem):
    c = lax.axis_index("c")
    s = lax.axis_index("s")
    base = (s * 2 + c) * _CROWS
    pltpu.sync_copy(selidx_hbm.at[pl.ds(base, _CROWS)], idx_v)
    pltpu.sync_copy(heavy_hbm.at[pl.ds(base, _CROWS)], rows_v)
    pltpu.async_copy(rows_v, out_ref.at[idx_v], sem).wait()


def _scatter_call(heavy, selidx, out_ref):
    mesh = plsc.VectorSubcoreMesh(core_axis_name="c", subcore_axis_name="s")
    f = pl.kernel(
        _scatter_body,
        out_type=[],
        mesh=mesh,
        scratch_types=[
            pltpu.VMEM((_CROWS,), jnp.int32),
            pltpu.VMEM((_CROWS, _DIM), jnp.float32),
            pltpu.SemaphoreType.DMA,
        ],
        compiler_params=pltpu.CompilerParams(needs_layout_passes=False),
    )
    f(heavy, selidx, out_ref)


def kernel(x, routing_token, gamma_l, w1_l, b1_l, w2_l, b2_l,
           gamma_h, w1_h, b1_h, w2_h, b2_h):
    xf = x.reshape(_NTOK, _DIM)
    selidx, gathered = _route_call(xf, routing_token)
    light = _light_call(xf, gamma_l, w1_l, b1_l, w2_l, b2_l)
    lightg = _lgather_call(light, selidx)
    heavy = _heavy_call(gathered, lightg, gamma_h, w1_h, b1_h, w2_h, b2_h)
    ref = jax.new_ref(light)
    _scatter_call(heavy, selidx, ref)
    return jax.freeze(ref).reshape(_B, _N, _DIM)
